# baseline (speedup 1.0000x reference)
import jax
import jax.numpy as jnp
from jax.experimental import pallas as pl
from jax.experimental.pallas import tpu as pltpu

_NEG_LOG2E = -1.4426950408889634


def _wattn_kernel(c, x_ref, m_ref, proj_ref, hid_ref, ev_ref, o_ref):
    x = x_ref[0]
    x8 = x.astype(jnp.float8_e4m3fn)
    p8 = proj_ref[...].astype(jnp.float8_e4m3fn)
    u0 = jnp.dot(x8, p8, preferred_element_type=jnp.float32)
    a = 1.0 / (1.0 + jnp.exp2(u0 * c))
    u = jax.lax.dot_general(hid_ref[0], a, (((0,), (1,)), ((), ())),
                            preferred_element_type=jnp.float32)
    a = 1.0 / (1.0 + jnp.exp2(u * c))
    for i in range(1, hid_ref.shape[0]):
        u = jax.lax.dot_general(hid_ref[i], a, (((0,), (0,)), ((), ())),
                                preferred_element_type=jnp.float32)
        a = 1.0 / (1.0 + jnp.exp2(u * c))
    u2 = jnp.sum(a * ev_ref[...], axis=0, keepdims=True) * c
    s = 1.0 / (1.0 + jnp.exp2(u2))
    e = jnp.exp(s)
    em = e * m_ref[0].astype(jnp.float32)
    em_col = em.T
    num = jax.lax.dot_general(em_col, x, (((0,), (0,)), ((), ())),
                              preferred_element_type=jnp.float32)
    den = jnp.sum(em)
    z = jnp.sum(e)
    o_ref[0] = num * (1.0 / (den + 1e-12 * z))


def kernel(inp, mask, projector, hidden, evaluator):
    B, S, D = inp.shape
    H = projector.shape[-1]
    c = _NEG_LOG2E / float(H) ** 0.5
    m2 = mask.reshape(B, 1, S)

    out = pl.pallas_call(
        lambda *refs: _wattn_kernel(c, *refs),
        grid=(B,),
        in_specs=[
            pl.BlockSpec((1, S, D), lambda b: (b, 0, 0)),
            pl.BlockSpec((1, 1, S), lambda b: (b, 0, 0)),
            pl.BlockSpec((D, H), lambda b: (0, 0)),
            pl.BlockSpec(hidden.shape, lambda b: (0, 0, 0)),
            pl.BlockSpec((H, 1), lambda b: (0, 0)),
        ],
        out_specs=pl.BlockSpec((1, 1, D), lambda b: (b, 0, 0)),
        out_shape=jax.ShapeDtypeStruct((B, 1, D), inp.dtype),
        compiler_params=pltpu.CompilerParams(
            dimension_semantics=("parallel",)),
    )(inp, m2, projector, hidden, evaluator)
    return out.reshape(B, D)
